# Initial kernel scaffold; baseline (speedup 1.0000x reference)
import jax
import jax.numpy as jnp
from jax.experimental import pallas as pl


def kernel(x, gamma):
    raise NotImplementedError("write your pallas kernel here")


# speedup vs baseline: 1.6583x; 1.6583x over previous
import functools

import jax
import jax.numpy as jnp
from jax import lax
from jax.experimental import pallas as pl
from jax.experimental.pallas import tpu as pltpu

BS = 7


def _threefry_bits(idx):
    rotations = ((13, 15, 26, 6), (17, 29, 16, 24))
    k0 = jnp.uint32(0)
    k1 = jnp.uint32(42)
    ks = (k0, k1, jnp.uint32(42 ^ 0x1BD11BDA))
    x0 = jnp.zeros_like(idx) + ks[0]
    x1 = idx + ks[1]
    for i in range(5):
        for r in rotations[i % 2]:
            x0 = x0 + x1
            x1 = lax.shift_left(x1, jnp.uint32(r)) | lax.shift_right_logical(
                x1, jnp.uint32(32 - r))
            x1 = x0 ^ x1
        x0 = x0 + ks[(i + 1) % 3]
        x1 = x1 + ks[(i + 2) % 3] + jnp.uint32(i + 1)
    return x0 ^ x1


def _win7_max(p, axis, out_len):
    def sl(a, start, length):
        idx = [slice(None)] * a.ndim
        idx[axis] = slice(start, start + length)
        return a[tuple(idx)]

    n = p.shape[axis]
    s1 = jnp.maximum(sl(p, 0, n - 1), sl(p, 1, n - 1))
    s2 = jnp.maximum(sl(s1, 0, n - 3), sl(s1, 2, n - 3))
    return jnp.maximum(sl(s2, 0, out_len), sl(s2, 3, out_len))


def _mask_kernel(gamma_ref, mask_ref, count_ref, *, G, mh, mw, H, W):
    step = pl.program_id(0)
    g = gamma_ref[0]

    plane = lax.broadcasted_iota(jnp.uint32, (G, mh, W), 0) + jnp.uint32(G) * step.astype(jnp.uint32)
    row = lax.broadcasted_iota(jnp.uint32, (G, mh, W), 1)
    col = lax.broadcasted_iota(jnp.uint32, (G, mh, W), 2)
    idx = plane * jnp.uint32(mh * mw) + row * jnp.uint32(mw) + col
    bits = _threefry_bits(idx)

    mant = lax.shift_right_logical(bits, jnp.uint32(9)).astype(jnp.float32)
    thresh = g * jnp.float32(8388608.0)
    mask = jnp.where((mant < thresh) & (col < mw), jnp.float32(1.0),
                     jnp.float32(0.0))

    zr = jnp.zeros((G, BS - 1, W), jnp.float32)
    zb = jnp.zeros((G, H - mh, W), jnp.float32)
    pr = jnp.concatenate([zr, mask, zb], axis=1)
    rm = _win7_max(pr, 1, H)
    zc = jnp.zeros((G, H, BS - 1), jnp.float32)
    pc = jnp.concatenate([zc, rm], axis=2)
    dl = _win7_max(pc, 2, W)

    bm = jnp.float32(1.0) - dl
    mask_ref[...] = bm.astype(jnp.int8)

    @pl.when(step == 0)
    def _init():
        count_ref[0, 0] = jnp.float32(0.0)

    count_ref[0, 0] += jnp.sum(bm)


def _scale_kernel(count_ref, x_ref, m_ref, o_ref, *, count_m):
    scale = jnp.float32(count_m) / count_ref[0, 0]
    o_ref[...] = x_ref[...] * m_ref[...].astype(jnp.float32) * scale


def kernel(x, gamma):
    B, C, H, W = x.shape
    mh, mw = H - (BS - 1), W - (BS - 1)
    nplanes = B * C
    count_m = nplanes * H * W

    G = 8
    mask_i8, count = pl.pallas_call(
        functools.partial(_mask_kernel, G=G, mh=mh, mw=mw, H=H, W=W),
        grid=(nplanes // G,),
        in_specs=[pl.BlockSpec(memory_space=pltpu.SMEM)],
        out_specs=[
            pl.BlockSpec((G, H, W), lambda i: (i, 0, 0)),
            pl.BlockSpec(memory_space=pltpu.SMEM),
        ],
        out_shape=[
            jax.ShapeDtypeStruct((nplanes, H, W), jnp.int8),
            jax.ShapeDtypeStruct((1, 1), jnp.float32),
        ],
    )(gamma)

    G2 = 8
    x3 = x.reshape(nplanes, H, W)
    out = pl.pallas_call(
        functools.partial(_scale_kernel, count_m=count_m),
        grid=(nplanes // G2,),
        in_specs=[
            pl.BlockSpec(memory_space=pltpu.SMEM),
            pl.BlockSpec((G2, H, W), lambda i: (i, 0, 0)),
            pl.BlockSpec((G2, H, W), lambda i: (i, 0, 0)),
        ],
        out_specs=pl.BlockSpec((G2, H, W), lambda i: (i, 0, 0)),
        out_shape=jax.ShapeDtypeStruct((nplanes, H, W), jnp.float32),
    )(count.reshape(1, 1), x3, mask_i8)

    return out.reshape(B, C, H, W)
